# baseline (speedup 1.0000x reference)
import functools

import jax
import jax.numpy as jnp
import numpy as np
from jax import lax
from jax.experimental import pallas as pl
from jax.experimental.pallas import tpu as pltpu
from jax.experimental.pallas import tpu_sc as plsc

_EPS = 1e-5
_NEG = -1e30
_BM_ROUTER = 1024
_BM = 512
_HT = 512
_BM_ADD = 512


def _router_body(x_ref, wr_ref, br_ref, i1_ref, i2_ref, w1_ref, w2_ref, *,
                 n_experts):
    logits = jnp.dot(x_ref[...], wr_ref[...].T,
                     preferred_element_type=jnp.float32) + br_ref[...]
    col = lax.broadcasted_iota(jnp.int32, logits.shape, 1)
    lm = jnp.where(col < n_experts, logits, _NEG)
    m1 = jnp.max(lm, axis=1, keepdims=True)
    i1 = jnp.min(jnp.where(lm == m1, col, 127), axis=1, keepdims=True)
    l2 = jnp.where(col == i1, _NEG, lm)
    m2 = jnp.max(l2, axis=1, keepdims=True)
    i2 = jnp.min(jnp.where(l2 == m2, col, 127), axis=1, keepdims=True)
    e = jnp.exp(m2 - m1)
    wa = 1.0 / (1.0 + e)
    i1_ref[...] = i1
    i2_ref[...] = i2
    w1_ref[...] = wa
    w2_ref[...] = 1.0 - wa


def _sc_gather_call(table, idx):
    rows_out = idx.shape[0]
    d = table.shape[1]
    nw = 32
    nbuf = 3
    bpw = rows_out // nw
    ch = 16
    nch = bpw // ch
    mesh = plsc.VectorSubcoreMesh(core_axis_name="c", subcore_axis_name="s")

    @functools.partial(
        pl.kernel, mesh=mesh,
        out_type=jax.ShapeDtypeStruct((rows_out, d), table.dtype),
        scratch_types=(
            [pltpu.VMEM((bpw,), jnp.int32)]
            + [pltpu.VMEM((ch, d), table.dtype) for _ in range(nbuf)]
            + [pltpu.SemaphoreType.DMA for _ in range(2 * nbuf)]
        ),
    )
    def gk(table_hbm, idx_hbm, out_hbm, idx_all, *bufs_sems):
        bufs = bufs_sems[:nbuf]
        sg = bufs_sems[nbuf:2 * nbuf]
        sw = bufs_sems[2 * nbuf:3 * nbuf]
        wid = lax.axis_index("s") * 2 + lax.axis_index("c")
        base = wid * bpw
        pltpu.sync_copy(idx_hbm.at[pl.ds(base, bpw)], idx_all)

        def start_gather(c):
            b = c % nbuf
            return pltpu.async_copy(
                table_hbm.at[idx_all.at[pl.ds(c * ch, ch)]], bufs[b], sg[b])

        hg = {}
        for c in range(min(nbuf, nch)):
            hg[c] = start_gather(c)
        pending_w = {}
        for c in range(nch):
            b = c % nbuf
            hg[c].wait()
            pending_w[c] = pltpu.async_copy(
                bufs[b], out_hbm.at[pl.ds(base + c * ch, ch)], sw[b])
            if c + nbuf < nch:
                pending_w.pop(c).wait()
                hg[c + nbuf] = start_gather(c + nbuf)
        for c in sorted(pending_w):
            pending_w[c].wait()

    return gk(table, idx)


def _ffn_body(wrow_ref, nh_ref, xrow_ref, act_ref, betab_ref,
              xg_ref, sw_ref, w1_ref, b1_ref, w2_ref, b2r_ref, o_ref):
    b = pl.program_id(0)
    h = pl.program_id(1)

    @pl.when((act_ref[b] == 1) & (h < nh_ref[b]))
    def _():
        hh = jnp.dot(xg_ref[...], w1_ref[...].T,
                     preferred_element_type=jnp.float32) + b1_ref[...]
        sw = sw_ref[...]
        hh = jnp.maximum(hh, 0.0) * sw
        contrib = jnp.dot(hh.astype(jnp.bfloat16), w2_ref[...],
                          preferred_element_type=jnp.float32)

        @pl.when(h == 0)
        def _():
            o_ref[...] = contrib + sw * b2r_ref[0]

        @pl.when(h > 0)
        def _():
            o_ref[...] += contrib


def _add_body(a_ref, b_ref, x_ref, o_ref):
    o_ref[...] = a_ref[...] + b_ref[...] + x_ref[...]


def kernel(x, router_W, router_b, expert_params):
    n, d = x.shape
    ne = len(expert_params)
    inv_c = 1.0 / np.sqrt(1.0 + _EPS)
    n2 = 2 * n
    p_slots = n2 + ne * _BM
    nblk = p_slots // _BM

    w1_parts, b1_parts, w2_parts, b2_rows = [], [], [], []
    nh_list = []
    for (W1, b1, g1, be1, W2, b2, g2, be2) in expert_params:
        s = W1.shape[0]
        sp = ((s + _HT - 1) // _HT) * _HT
        nh_list.append(sp // _HT)
        s1 = g1 * inv_c
        w1f = (W1 * s1[:, None]).astype(jnp.bfloat16)
        b1f = b1 * s1 + be1
        s2 = g2 * inv_c
        w2f = (W2 * s2[:, None]).T.astype(jnp.bfloat16)
        b2_rows.append(b2 * s2 + be2)
        w1_parts.append(jnp.pad(w1f, ((0, sp - s), (0, 0))))
        b1_parts.append(jnp.pad(b1f, (0, sp - s)))
        w2_parts.append(jnp.pad(w2f, ((0, sp - s), (0, 0))))
    w1cat = jnp.concatenate(w1_parts, axis=0)
    b1cat = jnp.concatenate(b1_parts, axis=0)[None, :]
    w2cat = jnp.concatenate(w2_parts, axis=0)
    b2cat = jnp.stack(b2_rows, axis=0)[:, None, :]
    nh_np = np.array(nh_list, dtype=np.int32)
    hofs_np = np.concatenate([[0], np.cumsum(nh_np)[:-1]]).astype(np.int32)
    nt = int(nh_np.max())

    wr_pad = jnp.pad(router_W, ((0, 128 - ne), (0, 0)))
    br_pad = jnp.pad(router_b, (0, 128 - ne))[None, :]

    bmr = min(_BM_ROUTER, n)
    nbr = n // bmr
    i1, i2, w1s, w2s = pl.pallas_call(
        functools.partial(_router_body, n_experts=ne),
        grid=(nbr,),
        in_specs=[
            pl.BlockSpec((bmr, d), lambda b: (b, 0)),
            pl.BlockSpec((128, d), lambda b: (0, 0)),
            pl.BlockSpec((1, 128), lambda b: (0, 0)),
        ],
        out_specs=[
            pl.BlockSpec((bmr, 1), lambda b: (b, 0)),
            pl.BlockSpec((bmr, 1), lambda b: (b, 0)),
            pl.BlockSpec((bmr, 1), lambda b: (b, 0)),
            pl.BlockSpec((bmr, 1), lambda b: (b, 0)),
        ],
        out_shape=[
            jax.ShapeDtypeStruct((n, 1), jnp.int32),
            jax.ShapeDtypeStruct((n, 1), jnp.int32),
            jax.ShapeDtypeStruct((n, 1), jnp.float32),
            jax.ShapeDtypeStruct((n, 1), jnp.float32),
        ],
    )(x, wr_pad, br_pad)

    e_flat = jnp.concatenate([i1[:, 0], i2[:, 0]])
    w_flat = jnp.concatenate([w1s[:, 0], w2s[:, 0]])
    order = jnp.argsort(e_flat, stable=True).astype(jnp.int32)
    e_sorted = jnp.take(e_flat, order)
    tok_sorted = jnp.mod(order, n).astype(jnp.int32)
    w_sorted = jnp.take(w_flat, order)
    counts = jnp.bincount(e_flat, length=ne).astype(jnp.int32)
    pc = ((counts + _BM - 1) // _BM) * _BM
    cum_pc = jnp.cumsum(pc)
    poff = cum_pc - pc
    cum_c = jnp.cumsum(counts)
    start = cum_c - counts
    r = jnp.arange(n2, dtype=jnp.int32)
    padpos = (jnp.take(poff, e_sorted) + (r - jnp.take(start, e_sorted))
              ).astype(jnp.int32)
    slot_tok = (jnp.arange(p_slots, dtype=jnp.int32) % n
                ).at[padpos].set(tok_sorted)
    slot_w = jnp.zeros((p_slots,), jnp.float32).at[padpos].set(w_sorted)
    posflat = jnp.zeros((n2,), jnp.int32).at[order].set(padpos)

    total_padded = cum_pc[-1]
    nblk_active = total_padded // _BM
    lab = nblk_active - 1
    blk = jnp.arange(nblk, dtype=jnp.int32)
    be0 = jnp.searchsorted(cum_pc, blk * _BM, side="right").astype(jnp.int32)
    active = blk < nblk_active
    be = jnp.where(active, be0, jnp.take(be0, lab))
    wrow = jnp.take(jnp.asarray(hofs_np), be).astype(jnp.int32)
    nh_b = jnp.take(jnp.asarray(nh_np), be).astype(jnp.int32)
    xrow = jnp.where(active, blk, lab).astype(jnp.int32)
    act = active.astype(jnp.int32)

    x_packed = lax.bitcast_convert_type(
        x.astype(jnp.bfloat16).reshape(n, d // 2, 2), jnp.int32)
    xg_packed = _sc_gather_call(x_packed, slot_tok)
    xg = lax.bitcast_convert_type(xg_packed[:, :, None],
                                  jnp.bfloat16).reshape(p_slots, d)

    grid_spec = pltpu.PrefetchScalarGridSpec(
        num_scalar_prefetch=5,
        grid=(nblk, nt),
        in_specs=[
            pl.BlockSpec((_BM, d), lambda b, h, wr, nh, xr, ac, b2: (xr[b], 0)),
            pl.BlockSpec((_BM, 1), lambda b, h, wr, nh, xr, ac, b2: (xr[b], 0)),
            pl.BlockSpec((_HT, d),
                         lambda b, h, wr, nh, xr, ac, b2:
                         (wr[b] + jnp.minimum(h, nh[b] - 1), 0)),
            pl.BlockSpec((1, _HT),
                         lambda b, h, wr, nh, xr, ac, b2:
                         (0, wr[b] + jnp.minimum(h, nh[b] - 1))),
            pl.BlockSpec((_HT, d),
                         lambda b, h, wr, nh, xr, ac, b2:
                         (wr[b] + jnp.minimum(h, nh[b] - 1), 0)),
            pl.BlockSpec((1, 1, d),
                         lambda b, h, wr, nh, xr, ac, b2: (b2[b], 0, 0)),
        ],
        out_specs=pl.BlockSpec((_BM, d),
                               lambda b, h, wr, nh, xr, ac, b2: (xr[b], 0)),
    )
    o_slots = pl.pallas_call(
        _ffn_body,
        grid_spec=grid_spec,
        out_shape=jax.ShapeDtypeStruct((p_slots, d), jnp.float32),
        compiler_params=pltpu.CompilerParams(
            dimension_semantics=("arbitrary", "arbitrary")),
    )(wrow, nh_b, xrow, act, be,
      xg, slot_w[:, None], w1cat, b1cat, w2cat, b2cat)

    g12 = _sc_gather_call(o_slots, posflat)

    bma = min(_BM_ADD, n)
    nba = n // bma
    out = pl.pallas_call(
        _add_body,
        grid=(nba,),
        in_specs=[
            pl.BlockSpec((bma, d), lambda b: (b, 0)),
            pl.BlockSpec((bma, d), lambda b: (b + nba, 0)),
            pl.BlockSpec((bma, d), lambda b: (b, 0)),
        ],
        out_specs=pl.BlockSpec((bma, d), lambda b: (b, 0)),
        out_shape=jax.ShapeDtypeStruct((n, d), jnp.float32),
    )(g12, g12, x)
    return out


# speedup vs baseline: 2.2798x; 2.2798x over previous
import functools

import jax
import jax.numpy as jnp
import numpy as np
from jax import lax
from jax.experimental import pallas as pl
from jax.experimental.pallas import tpu as pltpu

_EPS = 1e-5
_NEG = -1e30


def _ffn_body(etab_ref, x_ref, wr_ref, br_ref, b2c_ref, w1_ref, b1_ref,
              w2_ref, o_ref, coef_scr, x16_scr, *, n_experts):
    h_id = pl.program_id(1)

    @pl.when(h_id == 0)
    def _router():
        x = x_ref[...]
        x16_scr[...] = x.astype(jnp.bfloat16)
        logits = jnp.dot(x, wr_ref[...].T, preferred_element_type=jnp.float32)
        logits = logits + br_ref[...]
        col = lax.broadcasted_iota(jnp.int32, logits.shape, 1)
        lm = jnp.where(col < n_experts, logits, _NEG)
        m1 = jnp.max(lm, axis=1, keepdims=True)
        i1 = jnp.min(jnp.where(lm == m1, col, 127), axis=1, keepdims=True)
        l2 = jnp.where(col == i1, _NEG, lm)
        m2 = jnp.max(l2, axis=1, keepdims=True)
        i2 = jnp.min(jnp.where(l2 == m2, col, 127), axis=1, keepdims=True)
        e = jnp.exp(m2 - m1)
        wa = 1.0 / (1.0 + e)
        wb = 1.0 - wa
        coef = wa * (col == i1) + wb * (col == i2)
        coef_scr[...] = coef
        o_ref[...] = x + jnp.dot(coef, b2c_ref[...],
                                 preferred_element_type=jnp.float32)

    et = etab_ref[h_id]
    coef = coef_scr[...]
    col = lax.broadcasted_iota(jnp.int32, coef.shape, 1)
    csel = jnp.sum(jnp.where(col == et, coef, 0.0), axis=1)
    h = jnp.dot(x16_scr[...], w1_ref[...].T,
                preferred_element_type=jnp.float32)
    h = jnp.maximum(h + b1_ref[...], 0.0) * csel[:, None]
    o_ref[...] += jnp.dot(h.astype(jnp.bfloat16), w2_ref[...],
                          preferred_element_type=jnp.float32)


def kernel(x, router_W, router_b, expert_params):
    n, d = x.shape
    n_experts = len(expert_params)
    inv_c = 1.0 / np.sqrt(1.0 + _EPS)

    bm = min(1024, n)
    ht = 512

    w1_parts, b1_parts, w2_parts = [], [], []
    b2_rows = []
    sizes_p = []
    for (W1, b1, g1, be1, W2, b2, g2, be2) in expert_params:
        s = W1.shape[0]
        sp = ((s + ht - 1) // ht) * ht
        sizes_p.append(sp)
        s1 = g1 * inv_c
        w1f = (W1 * s1[:, None]).astype(jnp.bfloat16)
        b1f = b1 * s1 + be1
        s2 = g2 * inv_c
        w2f = (W2 * s2[:, None]).T.astype(jnp.bfloat16)
        b2f = b2 * s2 + be2
        w1_parts.append(jnp.pad(w1f, ((0, sp - s), (0, 0))))
        b1_parts.append(jnp.pad(b1f, (0, sp - s)))
        w2_parts.append(jnp.pad(w2f, ((0, sp - s), (0, 0))))
        b2_rows.append(b2f)

    w1cat = jnp.concatenate(w1_parts, axis=0)
    b1cat = jnp.concatenate(b1_parts, axis=0)[None, :]
    w2cat = jnp.concatenate(w2_parts, axis=0)
    s_tot = w1cat.shape[0]
    nt = s_tot // ht

    etab = np.repeat(np.arange(n_experts, dtype=np.int32),
                     [sp // ht for sp in sizes_p])

    wr_pad = jnp.pad(router_W, ((0, 128 - n_experts), (0, 0)))
    br_pad = jnp.pad(router_b, (0, 128 - n_experts))[None, :]
    b2c = jnp.pad(jnp.stack(b2_rows, axis=0), ((0, 128 - n_experts), (0, 0)))

    nb = n // bm
    grid_spec = pltpu.PrefetchScalarGridSpec(
        num_scalar_prefetch=1,
        grid=(nb, nt),
        in_specs=[
            pl.BlockSpec((bm, d), lambda b, h, tab: (b, 0)),
            pl.BlockSpec((128, d), lambda b, h, tab: (0, 0)),
            pl.BlockSpec((1, 128), lambda b, h, tab: (0, 0)),
            pl.BlockSpec((128, d), lambda b, h, tab: (0, 0)),
            pl.BlockSpec((ht, d), lambda b, h, tab: (h, 0)),
            pl.BlockSpec((1, ht), lambda b, h, tab: (0, h)),
            pl.BlockSpec((ht, d), lambda b, h, tab: (h, 0)),
        ],
        out_specs=pl.BlockSpec((bm, d), lambda b, h, tab: (b, 0)),
        scratch_shapes=[pltpu.VMEM((bm, 128), jnp.float32),
                        pltpu.VMEM((bm, d), jnp.bfloat16)],
    )

    out = pl.pallas_call(
        functools.partial(_ffn_body, n_experts=n_experts),
        grid_spec=grid_spec,
        out_shape=jax.ShapeDtypeStruct((n, d), jnp.float32),
        compiler_params=pltpu.CompilerParams(
            dimension_semantics=("arbitrary", "arbitrary")),
    )(jnp.asarray(etab), x, wr_pad, br_pad, b2c, w1cat, b1cat, w2cat)
    return out
